# baseline (speedup 1.0000x reference)
import functools

import jax
import jax.numpy as jnp
from jax import lax
from jax.experimental import pallas as pl
from jax.experimental.pallas import tpu as pltpu
from jax.experimental.pallas import tpu_sc as plsc

N_SPARSE = 26
N_SEQ = 2
B = 4096
EMBED = 32
SEQ_LEN = 50
NW = 32
BPW = B // NW
D = 6
K = 3
DS = 4
INV_LEN = 1.0 / SEQ_LEN


def _sc_body(*refs):
    idx_all = refs[0]
    seq_idx = refs[1:3]
    big_table = refs[3]
    seq_tables = refs[4:6]
    outs = refs[6:6 + N_SPARSE]
    seq_outs = refs[32:34]
    idxbuf = refs[34]
    idxseq = refs[35]
    rowsbuf = refs[36]
    srows = refs[37]
    pooled = refs[38]
    gsems = refs[39:39 + D]
    ssems = refs[39 + D:39 + 2 * D]
    qsems = refs[39 + 2 * D:39 + 2 * D + DS]

    wid = lax.axis_index("s") * 2 + lax.axis_index("c")
    base = wid * BPW

    pltpu.sync_copy(idx_all.at[wid], idxbuf)

    def fire_gather(i):
        b = i % D
        pltpu.async_copy(big_table.at[idxbuf.at[i]], rowsbuf.at[b], gsems[b])

    def wait_gather(i):
        b = i % D
        pltpu.make_async_copy(
            big_table.at[idxbuf.at[i]], rowsbuf.at[b], gsems[b]).wait()

    def fire_store(i):
        b = i % D
        pltpu.async_copy(rowsbuf.at[b], outs[i].at[pl.ds(base, BPW)], ssems[b])

    def wait_store(i):
        b = i % D
        pltpu.make_async_copy(
            rowsbuf.at[b], outs[i].at[pl.ds(base, BPW)], ssems[b]).wait()

    for i in range(K):
        fire_gather(i)
    for i in range(N_SPARSE):
        wait_gather(i)
        fire_store(i)
        j = i + K
        if j < N_SPARSE:
            if j - D >= 0:
                wait_store(j - D)
            fire_gather(j)

    zero = jnp.zeros((16,), jnp.float32)

    for f in range(N_SEQ):
        pltpu.sync_copy(seq_idx[f].at[pl.ds(base, BPW)], idxseq)

        def sfire(s, d):
            pltpu.async_copy(
                seq_tables[f].at[idxseq.at[s]], srows.at[d], qsems[d])

        def swait(s, d):
            pltpu.make_async_copy(
                seq_tables[f].at[idxseq.at[s]], srows.at[d], qsems[d]).wait()

        for d in range(DS):
            sfire(d, d)

        def outer(g, _):
            for d in range(DS):
                s = g * DS + d
                swait(s, d)

                def acc(j, carry):
                    a0, a1 = carry
                    a0 = a0 + srows[d, j, 0:16]
                    a1 = a1 + srows[d, j, 16:32]
                    return (a0, a1)

                a0, a1 = lax.fori_loop(0, SEQ_LEN, acc, (zero, zero))
                pooled[s, 0:16] = a0 * INV_LEN
                pooled[s, 16:32] = a1 * INV_LEN

                @pl.when(s + DS < BPW)
                def _():
                    sfire(s + DS, d)
            return 0

        lax.fori_loop(0, BPW // DS, outer, 0)
        pltpu.sync_copy(pooled, seq_outs[f].at[pl.ds(base, BPW)])

    for i in range(N_SPARSE - D, N_SPARSE):
        wait_store(i)


@functools.lru_cache(maxsize=None)
def _make_sc_call():
    mesh = plsc.VectorSubcoreMesh(core_axis_name="c", subcore_axis_name="s")
    out_type = [jax.ShapeDtypeStruct((B, EMBED), jnp.float32)] * (N_SPARSE + N_SEQ)
    scratch_types = [
        pltpu.VMEM((N_SPARSE, BPW), jnp.int32),
        pltpu.VMEM((BPW, SEQ_LEN), jnp.int32),
        pltpu.VMEM((D, BPW, EMBED), jnp.float32),
        pltpu.VMEM((DS, SEQ_LEN, EMBED), jnp.float32),
        pltpu.VMEM((BPW, EMBED), jnp.float32),
    ] + [pltpu.SemaphoreType.DMA] * (2 * D + DS)
    return functools.partial(
        pl.kernel, mesh=mesh, out_type=out_type, scratch_types=scratch_types,
        compiler_params=pltpu.CompilerParams(use_tc_tiling_on_sc=False),
    )(_sc_body)


def _bn_body(x_ref, g_ref, b_ref, o_ref):
    x = x_ref[...]
    mean = jnp.mean(x, axis=0, keepdims=True)
    xc = x - mean
    var = jnp.mean(xc * xc, axis=0, keepdims=True)
    o_ref[...] = xc * lax.rsqrt(var + 1e-5) * g_ref[...] + b_ref[...]


_bn_call = pl.pallas_call(
    _bn_body,
    out_shape=jax.ShapeDtypeStruct((B, 4), jnp.float32),
)


def kernel(sparse_0, sparse_1, sparse_2, sparse_3, sparse_4, sparse_5, sparse_6, sparse_7, sparse_8, sparse_9, sparse_10, sparse_11, sparse_12, sparse_13, sparse_14, sparse_15, sparse_16, sparse_17, sparse_18, sparse_19, sparse_20, sparse_21, sparse_22, sparse_23, sparse_24, sparse_25, dense_0, dense_1, dense_2, dense_3, seq_0, seq_1, sparse_table_0, sparse_table_1, sparse_table_2, sparse_table_3, sparse_table_4, sparse_table_5, sparse_table_6, sparse_table_7, sparse_table_8, sparse_table_9, sparse_table_10, sparse_table_11, sparse_table_12, sparse_table_13, sparse_table_14, sparse_table_15, sparse_table_16, sparse_table_17, sparse_table_18, sparse_table_19, sparse_table_20, sparse_table_21, sparse_table_22, sparse_table_23, sparse_table_24, sparse_table_25, seq_table_0, seq_table_1, bn_gamma_0, bn_gamma_1, bn_gamma_2, bn_gamma_3, bn_beta_0, bn_beta_1, bn_beta_2, bn_beta_3):
    sparse_ids = [sparse_0, sparse_1, sparse_2, sparse_3, sparse_4, sparse_5,
                  sparse_6, sparse_7, sparse_8, sparse_9, sparse_10, sparse_11,
                  sparse_12, sparse_13, sparse_14, sparse_15, sparse_16,
                  sparse_17, sparse_18, sparse_19, sparse_20, sparse_21,
                  sparse_22, sparse_23, sparse_24, sparse_25]
    tables = [sparse_table_0, sparse_table_1, sparse_table_2, sparse_table_3,
              sparse_table_4, sparse_table_5, sparse_table_6, sparse_table_7,
              sparse_table_8, sparse_table_9, sparse_table_10, sparse_table_11,
              sparse_table_12, sparse_table_13, sparse_table_14,
              sparse_table_15, sparse_table_16, sparse_table_17,
              sparse_table_18, sparse_table_19, sparse_table_20,
              sparse_table_21, sparse_table_22, sparse_table_23,
              sparse_table_24, sparse_table_25]

    big_table = jnp.concatenate(tables, axis=0)
    offs = (jnp.arange(N_SPARSE, dtype=jnp.int32) * tables[0].shape[0])[:, None]
    idx_all = jnp.stack([s.astype(jnp.int32) for s in sparse_ids], axis=0) + offs
    idx_all = idx_all.reshape(N_SPARSE, NW, BPW).transpose(1, 0, 2)

    sc_outs = _make_sc_call()(
        idx_all, seq_0.astype(jnp.int32), seq_1.astype(jnp.int32),
        big_table, seq_table_0, seq_table_1)
    sparse_outs = sc_outs[:N_SPARSE]
    seq_outs = sc_outs[N_SPARSE:]

    x = jnp.concatenate([dense_0, dense_1, dense_2, dense_3], axis=1)
    g = jnp.stack([bn_gamma_0[0], bn_gamma_1[0], bn_gamma_2[0], bn_gamma_3[0]]).reshape(1, 4)
    bta = jnp.stack([bn_beta_0[0], bn_beta_1[0], bn_beta_2[0], bn_beta_3[0]]).reshape(1, 4)
    y = _bn_call(x, g, bta)
    dense_outs = [y[:, i:i + 1] for i in range(4)]

    return tuple(sparse_outs) + tuple(dense_outs) + tuple(seq_outs)


# speedup vs baseline: 1.4618x; 1.4618x over previous
import functools

import jax
import jax.numpy as jnp
from jax import lax
from jax.experimental import pallas as pl
from jax.experimental.pallas import tpu as pltpu
from jax.experimental.pallas import tpu_sc as plsc

N_SPARSE = 26
N_SEQ = 2
B = 4096
EMBED = 32
SEQ_LEN = 50
NW = 32
BPW = B // NW
D = 6
K = 3
DS = 4
INV_LEN = 1.0 / SEQ_LEN


def _sc_body(*refs):
    sp_idx = refs[0:N_SPARSE]
    seq_idx = refs[26:28]
    tables = refs[28:28 + N_SPARSE]
    seq_tables = refs[54:56]
    outs = refs[56:56 + N_SPARSE]
    seq_outs = refs[82:84]
    idxbuf = refs[84]
    idxseq = refs[85]
    rowsbuf = refs[86]
    srows = refs[87]
    pooled = refs[88]
    isems = refs[89:89 + K]
    gsems = refs[89 + K:89 + K + D]
    ssems = refs[89 + K + D:89 + K + 2 * D]
    qsems = refs[89 + K + 2 * D:89 + K + 2 * D + DS]

    wid = lax.axis_index("s") * 2 + lax.axis_index("c")
    base = wid * BPW

    def fire_idx(i):
        pltpu.async_copy(sp_idx[i].at[pl.ds(base, BPW)], idxbuf.at[i],
                         isems[i % K])

    def wait_idx(i):
        pltpu.make_async_copy(sp_idx[i].at[pl.ds(base, BPW)], idxbuf.at[i],
                              isems[i % K]).wait()

    def fire_gather(i):
        b = i % D
        pltpu.async_copy(tables[i].at[idxbuf.at[i]], rowsbuf.at[b], gsems[b])

    def wait_gather(i):
        b = i % D
        pltpu.make_async_copy(
            tables[i].at[idxbuf.at[i]], rowsbuf.at[b], gsems[b]).wait()

    def fire_store(i):
        b = i % D
        pltpu.async_copy(rowsbuf.at[b], outs[i].at[pl.ds(base, BPW)], ssems[b])

    def wait_store(i):
        b = i % D
        pltpu.make_async_copy(
            rowsbuf.at[b], outs[i].at[pl.ds(base, BPW)], ssems[b]).wait()

    for j in range(K):
        fire_idx(j)
    for j in range(K):
        wait_idx(j)
        fire_gather(j)
        if j + K < N_SPARSE:
            fire_idx(j + K)
    for i in range(N_SPARSE):
        wait_gather(i)
        fire_store(i)
        j = i + K
        if j < N_SPARSE:
            if j - D >= 0:
                wait_store(j - D)
            wait_idx(j)
            fire_gather(j)
            if j + K < N_SPARSE:
                fire_idx(j + K)

    zero = jnp.zeros((16,), jnp.float32)

    for f in range(N_SEQ):
        pltpu.sync_copy(seq_idx[f].at[pl.ds(base, BPW)], idxseq)

        def sfire(s, d):
            pltpu.async_copy(
                seq_tables[f].at[idxseq.at[s]], srows.at[d], qsems[d])

        def swait(s, d):
            pltpu.make_async_copy(
                seq_tables[f].at[idxseq.at[s]], srows.at[d], qsems[d]).wait()

        for d in range(DS):
            sfire(d, d)

        def outer(g, _):
            for d in range(DS):
                s = g * DS + d
                swait(s, d)

                def acc(j, carry):
                    a0, a1 = carry
                    a0 = a0 + srows[d, j, 0:16]
                    a1 = a1 + srows[d, j, 16:32]
                    return (a0, a1)

                a0, a1 = lax.fori_loop(0, SEQ_LEN, acc, (zero, zero))
                pooled[s, 0:16] = a0 * INV_LEN
                pooled[s, 16:32] = a1 * INV_LEN

                @pl.when(s + DS < BPW)
                def _():
                    sfire(s + DS, d)
            return 0

        lax.fori_loop(0, BPW // DS, outer, 0)
        pltpu.sync_copy(pooled, seq_outs[f].at[pl.ds(base, BPW)])

    for i in range(N_SPARSE - D, N_SPARSE):
        wait_store(i)


@functools.lru_cache(maxsize=None)
def _make_sc_call():
    mesh = plsc.VectorSubcoreMesh(core_axis_name="c", subcore_axis_name="s")
    out_type = [jax.ShapeDtypeStruct((B, EMBED), jnp.float32)] * (N_SPARSE + N_SEQ)
    scratch_types = [
        pltpu.VMEM((N_SPARSE, BPW), jnp.int32),
        pltpu.VMEM((BPW, SEQ_LEN), jnp.int32),
        pltpu.VMEM((D, BPW, EMBED), jnp.float32),
        pltpu.VMEM((DS, SEQ_LEN, EMBED), jnp.float32),
        pltpu.VMEM((BPW, EMBED), jnp.float32),
    ] + [pltpu.SemaphoreType.DMA] * (K + 2 * D + DS)
    return functools.partial(
        pl.kernel, mesh=mesh, out_type=out_type, scratch_types=scratch_types,
        compiler_params=pltpu.CompilerParams(use_tc_tiling_on_sc=False),
    )(_sc_body)


def _bn_body(x_ref, g_ref, b_ref, o_ref):
    x = x_ref[...]
    mean = jnp.mean(x, axis=0, keepdims=True)
    xc = x - mean
    var = jnp.mean(xc * xc, axis=0, keepdims=True)
    o_ref[...] = xc * lax.rsqrt(var + 1e-5) * g_ref[...] + b_ref[...]


_bn_call = pl.pallas_call(
    _bn_body,
    out_shape=jax.ShapeDtypeStruct((B, 4), jnp.float32),
)


def kernel(sparse_0, sparse_1, sparse_2, sparse_3, sparse_4, sparse_5, sparse_6, sparse_7, sparse_8, sparse_9, sparse_10, sparse_11, sparse_12, sparse_13, sparse_14, sparse_15, sparse_16, sparse_17, sparse_18, sparse_19, sparse_20, sparse_21, sparse_22, sparse_23, sparse_24, sparse_25, dense_0, dense_1, dense_2, dense_3, seq_0, seq_1, sparse_table_0, sparse_table_1, sparse_table_2, sparse_table_3, sparse_table_4, sparse_table_5, sparse_table_6, sparse_table_7, sparse_table_8, sparse_table_9, sparse_table_10, sparse_table_11, sparse_table_12, sparse_table_13, sparse_table_14, sparse_table_15, sparse_table_16, sparse_table_17, sparse_table_18, sparse_table_19, sparse_table_20, sparse_table_21, sparse_table_22, sparse_table_23, sparse_table_24, sparse_table_25, seq_table_0, seq_table_1, bn_gamma_0, bn_gamma_1, bn_gamma_2, bn_gamma_3, bn_beta_0, bn_beta_1, bn_beta_2, bn_beta_3):
    sparse_ids = [sparse_0, sparse_1, sparse_2, sparse_3, sparse_4, sparse_5,
                  sparse_6, sparse_7, sparse_8, sparse_9, sparse_10, sparse_11,
                  sparse_12, sparse_13, sparse_14, sparse_15, sparse_16,
                  sparse_17, sparse_18, sparse_19, sparse_20, sparse_21,
                  sparse_22, sparse_23, sparse_24, sparse_25]
    tables = [sparse_table_0, sparse_table_1, sparse_table_2, sparse_table_3,
              sparse_table_4, sparse_table_5, sparse_table_6, sparse_table_7,
              sparse_table_8, sparse_table_9, sparse_table_10, sparse_table_11,
              sparse_table_12, sparse_table_13, sparse_table_14,
              sparse_table_15, sparse_table_16, sparse_table_17,
              sparse_table_18, sparse_table_19, sparse_table_20,
              sparse_table_21, sparse_table_22, sparse_table_23,
              sparse_table_24, sparse_table_25]

    sc_outs = _make_sc_call()(
        *[s.astype(jnp.int32) for s in sparse_ids],
        seq_0.astype(jnp.int32), seq_1.astype(jnp.int32),
        *tables, seq_table_0, seq_table_1)
    sparse_outs = sc_outs[:N_SPARSE]
    seq_outs = sc_outs[N_SPARSE:]

    x = jnp.concatenate([dense_0, dense_1, dense_2, dense_3], axis=1)
    g = jnp.stack([bn_gamma_0[0], bn_gamma_1[0], bn_gamma_2[0], bn_gamma_3[0]]).reshape(1, 4)
    bta = jnp.stack([bn_beta_0[0], bn_beta_1[0], bn_beta_2[0], bn_beta_3[0]]).reshape(1, 4)
    y = _bn_call(x, g, bta)
    dense_outs = [y[:, i:i + 1] for i in range(4)]

    return tuple(sparse_outs) + tuple(dense_outs) + tuple(seq_outs)


# speedup vs baseline: 1.4885x; 1.0183x over previous
import functools

import jax
import jax.numpy as jnp
from jax import lax
from jax.experimental import pallas as pl
from jax.experimental.pallas import tpu as pltpu
from jax.experimental.pallas import tpu_sc as plsc

N_SPARSE = 26
N_SEQ = 2
B = 4096
EMBED = 32
SEQ_LEN = 50
NW = 32
BPW = B // NW
D = 6
K = 3
DS = 4
SG = 8
INV_LEN = 1.0 / SEQ_LEN


def _sc_body(*refs):
    sp_idx = refs[0:N_SPARSE]
    seq_idx = refs[26:28]
    tables = refs[28:28 + N_SPARSE]
    seq_tables = refs[54:56]
    outs = refs[56:56 + N_SPARSE]
    seq_outs = refs[82:84]
    idxbuf = refs[84]
    idxseq = refs[85]
    rowsbuf = refs[86]
    srows = refs[87]
    pooled = refs[88]
    isems = refs[89:89 + K]
    gsems = refs[89 + K:89 + K + D]
    ssems = refs[89 + K + D:89 + K + 2 * D]
    qsems = refs[89 + K + 2 * D:89 + K + 2 * D + DS]

    wid = lax.axis_index("s") * 2 + lax.axis_index("c")
    base = wid * BPW

    def fire_idx(i):
        pltpu.async_copy(sp_idx[i].at[pl.ds(base, BPW)], idxbuf.at[i],
                         isems[i % K])

    def wait_idx(i):
        pltpu.make_async_copy(sp_idx[i].at[pl.ds(base, BPW)], idxbuf.at[i],
                              isems[i % K]).wait()

    def fire_gather(i):
        b = i % D
        pltpu.async_copy(tables[i].at[idxbuf.at[i]], rowsbuf.at[b], gsems[b])

    def wait_gather(i):
        b = i % D
        pltpu.make_async_copy(
            tables[i].at[idxbuf.at[i]], rowsbuf.at[b], gsems[b]).wait()

    def fire_store(i):
        b = i % D
        pltpu.async_copy(rowsbuf.at[b], outs[i].at[pl.ds(base, BPW)], ssems[b])

    def wait_store(i):
        b = i % D
        pltpu.make_async_copy(
            rowsbuf.at[b], outs[i].at[pl.ds(base, BPW)], ssems[b]).wait()

    for j in range(K):
        fire_idx(j)
    for j in range(K):
        wait_idx(j)
        fire_gather(j)
        if j + K < N_SPARSE:
            fire_idx(j + K)
    for i in range(N_SPARSE):
        wait_gather(i)
        fire_store(i)
        j = i + K
        if j < N_SPARSE:
            if j - D >= 0:
                wait_store(j - D)
            wait_idx(j)
            fire_gather(j)
            if j + K < N_SPARSE:
                fire_idx(j + K)

    zero = jnp.zeros((16,), jnp.float32)
    n_chunks = BPW // SG

    for f in range(N_SEQ):
        pltpu.sync_copy(seq_idx[f].at[pl.ds(wid * n_chunks, n_chunks)], idxseq)

        def sfire(g, d):
            pltpu.async_copy(
                seq_tables[f].at[idxseq.at[g]], srows.at[d], qsems[d])

        def swait(g, d):
            pltpu.make_async_copy(
                seq_tables[f].at[idxseq.at[g]], srows.at[d], qsems[d]).wait()

        for d in range(DS):
            sfire(d, d)

        def souter(gg, _):
            for d in range(DS):
                g = gg * DS + d
                swait(g, d)

                def sacc(k, _):
                    a0 = zero
                    a1 = zero
                    r0 = k * SEQ_LEN
                    for j in range(SEQ_LEN):
                        a0 = a0 + srows[d, r0 + j, 0:16]
                        a1 = a1 + srows[d, r0 + j, 16:32]
                    s = g * SG + k
                    pooled[s, 0:16] = a0 * INV_LEN
                    pooled[s, 16:32] = a1 * INV_LEN
                    return 0

                lax.fori_loop(0, SG, sacc, 0)

                @pl.when(g + DS < n_chunks)
                def _():
                    sfire(g + DS, d)
            return 0

        lax.fori_loop(0, n_chunks // DS, souter, 0)
        pltpu.sync_copy(pooled, seq_outs[f].at[pl.ds(base, BPW)])

    for i in range(N_SPARSE - D, N_SPARSE):
        wait_store(i)


@functools.lru_cache(maxsize=None)
def _make_sc_call():
    mesh = plsc.VectorSubcoreMesh(core_axis_name="c", subcore_axis_name="s")
    out_type = [jax.ShapeDtypeStruct((B, EMBED), jnp.float32)] * (N_SPARSE + N_SEQ)
    scratch_types = [
        pltpu.VMEM((N_SPARSE, BPW), jnp.int32),
        pltpu.VMEM((BPW // SG, SG * SEQ_LEN), jnp.int32),
        pltpu.VMEM((D, BPW, EMBED), jnp.float32),
        pltpu.VMEM((DS, SG * SEQ_LEN, EMBED), jnp.float32),
        pltpu.VMEM((BPW, EMBED), jnp.float32),
    ] + [pltpu.SemaphoreType.DMA] * (K + 2 * D + DS)
    return functools.partial(
        pl.kernel, mesh=mesh, out_type=out_type, scratch_types=scratch_types,
        compiler_params=pltpu.CompilerParams(use_tc_tiling_on_sc=False),
    )(_sc_body)


def _bn_body(x_ref, g_ref, b_ref, o_ref):
    x = x_ref[...]
    mean = jnp.mean(x, axis=0, keepdims=True)
    xc = x - mean
    var = jnp.mean(xc * xc, axis=0, keepdims=True)
    o_ref[...] = xc * lax.rsqrt(var + 1e-5) * g_ref[...] + b_ref[...]


_bn_call = pl.pallas_call(
    _bn_body,
    out_shape=jax.ShapeDtypeStruct((B, 4), jnp.float32),
)


def kernel(sparse_0, sparse_1, sparse_2, sparse_3, sparse_4, sparse_5, sparse_6, sparse_7, sparse_8, sparse_9, sparse_10, sparse_11, sparse_12, sparse_13, sparse_14, sparse_15, sparse_16, sparse_17, sparse_18, sparse_19, sparse_20, sparse_21, sparse_22, sparse_23, sparse_24, sparse_25, dense_0, dense_1, dense_2, dense_3, seq_0, seq_1, sparse_table_0, sparse_table_1, sparse_table_2, sparse_table_3, sparse_table_4, sparse_table_5, sparse_table_6, sparse_table_7, sparse_table_8, sparse_table_9, sparse_table_10, sparse_table_11, sparse_table_12, sparse_table_13, sparse_table_14, sparse_table_15, sparse_table_16, sparse_table_17, sparse_table_18, sparse_table_19, sparse_table_20, sparse_table_21, sparse_table_22, sparse_table_23, sparse_table_24, sparse_table_25, seq_table_0, seq_table_1, bn_gamma_0, bn_gamma_1, bn_gamma_2, bn_gamma_3, bn_beta_0, bn_beta_1, bn_beta_2, bn_beta_3):
    sparse_ids = [sparse_0, sparse_1, sparse_2, sparse_3, sparse_4, sparse_5,
                  sparse_6, sparse_7, sparse_8, sparse_9, sparse_10, sparse_11,
                  sparse_12, sparse_13, sparse_14, sparse_15, sparse_16,
                  sparse_17, sparse_18, sparse_19, sparse_20, sparse_21,
                  sparse_22, sparse_23, sparse_24, sparse_25]
    tables = [sparse_table_0, sparse_table_1, sparse_table_2, sparse_table_3,
              sparse_table_4, sparse_table_5, sparse_table_6, sparse_table_7,
              sparse_table_8, sparse_table_9, sparse_table_10, sparse_table_11,
              sparse_table_12, sparse_table_13, sparse_table_14,
              sparse_table_15, sparse_table_16, sparse_table_17,
              sparse_table_18, sparse_table_19, sparse_table_20,
              sparse_table_21, sparse_table_22, sparse_table_23,
              sparse_table_24, sparse_table_25]

    sc_outs = _make_sc_call()(
        *[s.astype(jnp.int32) for s in sparse_ids],
        seq_0.astype(jnp.int32).reshape(B // SG, SG * SEQ_LEN),
        seq_1.astype(jnp.int32).reshape(B // SG, SG * SEQ_LEN),
        *tables, seq_table_0, seq_table_1)
    sparse_outs = sc_outs[:N_SPARSE]
    seq_outs = sc_outs[N_SPARSE:]

    x = jnp.concatenate([dense_0, dense_1, dense_2, dense_3], axis=1)
    g = jnp.stack([bn_gamma_0[0], bn_gamma_1[0], bn_gamma_2[0], bn_gamma_3[0]]).reshape(1, 4)
    bta = jnp.stack([bn_beta_0[0], bn_beta_1[0], bn_beta_2[0], bn_beta_3[0]]).reshape(1, 4)
    y = _bn_call(x, g, bta)
    dense_outs = [y[:, i:i + 1] for i in range(4)]

    return tuple(sparse_outs) + tuple(dense_outs) + tuple(seq_outs)
